# baseline (speedup 1.0000x reference)
import functools

import jax
import jax.numpy as jnp
from jax import lax
from jax.experimental import pallas as pl
from jax.experimental.pallas import tpu as pltpu
from jax.experimental.pallas import tpu_sc as plsc

DICT_LEN = 1000000
LATENT_SIZE = 32

S = 200
B = 16384
G = 128
K = 4
KG = K * G

NC = 2
NS = 16
NW = NC * NS


def _embed_kernel(n_stages, idx_hbm, table_hbm, out_hbm,
                  idx_v, rows_v, trans_v, gsem0, gsem1, wsem0, wsem1):
    wid = lax.axis_index("s") * NC + lax.axis_index("c")
    flat_base = wid * n_stages * KG
    gsems = (gsem0, gsem1)
    wsems = (wsem0, wsem1)

    iota = lax.iota(jnp.int32, 16)
    bvecs = [iota + (g * 16) for g in range(8)]

    def stage_and_fire(a, t):
        f0 = flat_base + t * KG
        pltpu.sync_copy(idx_hbm.at[pl.ds(f0, KG)], idx_v.at[a])
        for j in range(K):
            pltpu.async_copy(
                table_hbm.at[idx_v.at[a, pl.ds(j * G, G)]],
                rows_v.at[a, pl.ds(j * G, G)],
                gsems[a],
            )

    def drain_gathers(a):
        for j in range(K):
            pltpu.make_async_copy(
                table_hbm.at[pl.ds(0, G)],
                rows_v.at[a, pl.ds(j * G, G)],
                gsems[a],
            ).wait()

    def transpose(a, c):
        def tr_body(j, carry):
            b_base = j * G
            for d in range(LATENT_SIZE):
                dvec = jnp.full((16,), d, jnp.int32)
                for g in range(8):
                    v = plsc.load_gather(
                        rows_v.at[a], [bvecs[g] + b_base, dvec]
                    )
                    trans_v[c, d, pl.ds(b_base + g * 16, 16)] = v
            return carry

        lax.fori_loop(0, K, tr_body, 0)

    def fire_write(c, t):
        f0 = flat_base + t * KG
        s2 = f0 // B
        b0 = f0 % B
        pltpu.async_copy(
            trans_v.at[c],
            out_hbm.at[pl.ds(s2 * LATENT_SIZE, LATENT_SIZE), pl.ds(b0, KG)],
            wsems[c],
        )

    def drain_write(c):
        pltpu.make_async_copy(
            out_hbm.at[pl.ds(0, LATENT_SIZE), pl.ds(0, KG)],
            trans_v.at[c],
            wsems[c],
        ).wait()

    stage_and_fire(0, 0)
    stage_and_fire(1, 1)
    drain_gathers(0)
    transpose(0, 0)
    fire_write(0, 0)
    stage_and_fire(0, 2)
    drain_gathers(1)
    transpose(1, 1)
    fire_write(1, 1)

    def body(p, carry):
        t = 2 * p
        stage_and_fire(1, t + 1)
        drain_gathers(0)
        drain_write(0)
        transpose(0, 0)
        fire_write(0, t)
        stage_and_fire(0, t + 2)
        drain_gathers(1)
        drain_write(1)
        transpose(1, 1)
        fire_write(1, t + 1)
        return carry

    lax.fori_loop(1, n_stages // 2 - 1, body, 0)

    t = n_stages - 2
    stage_and_fire(1, t + 1)
    drain_gathers(0)
    drain_write(0)
    transpose(0, 0)
    fire_write(0, t)
    drain_gathers(1)
    drain_write(1)
    transpose(1, 1)
    fire_write(1, t + 1)
    drain_write(0)
    drain_write(1)


def kernel(x, latent_tdirs):
    xt = jnp.transpose(x, (1, 2, 0)).reshape(S * B).astype(jnp.int32)
    n_stages = (S * B) // KG // NW

    mesh = plsc.VectorSubcoreMesh(core_axis_name="c", subcore_axis_name="s")
    run = functools.partial(
        pl.kernel,
        mesh=mesh,
        compiler_params=pltpu.CompilerParams(
            use_tc_tiling_on_sc=False, needs_layout_passes=False
        ),
        out_type=jax.ShapeDtypeStruct((S * LATENT_SIZE, B), jnp.float32),
        scratch_types=[
            pltpu.VMEM((2, KG), jnp.int32),
            pltpu.VMEM((2, KG, LATENT_SIZE), jnp.float32),
            pltpu.VMEM((2, LATENT_SIZE, KG), jnp.float32),
            pltpu.SemaphoreType.DMA,
            pltpu.SemaphoreType.DMA,
            pltpu.SemaphoreType.DMA,
            pltpu.SemaphoreType.DMA,
        ],
    )(functools.partial(_embed_kernel, n_stages))

    out_t = run(xt, latent_tdirs)
    return jnp.transpose(out_t.reshape(S, LATENT_SIZE, B), (2, 0, 1))


# speedup vs baseline: 2.4778x; 2.4778x over previous
import functools

import jax
import jax.numpy as jnp
from jax import lax
from jax.experimental import pallas as pl
from jax.experimental.pallas import tpu as pltpu
from jax.experimental.pallas import tpu_sc as plsc

DICT_LEN = 1000000
LATENT_SIZE = 32

S = 200
B = 16384
G = 128
K = 4
KG = K * G

NC = 2
NS = 16
NW = NC * NS


def _embed_kernel(n_stages, idx_hbm, table_hbm, out_hbm,
                  idx_v, rows_v, trans_v, gsem0, gsem1, wsem0, wsem1):
    wid = lax.axis_index("s") * NC + lax.axis_index("c")
    flat_base = wid * n_stages * KG
    gsems = (gsem0, gsem1)
    wsems = (wsem0, wsem1)

    iota = lax.iota(jnp.int32, 16)

    def stage_and_fire(a, t):
        f0 = flat_base + t * KG
        pltpu.sync_copy(idx_hbm.at[pl.ds(f0, KG)], idx_v.at[a])
        for j in range(K):
            pltpu.async_copy(
                table_hbm.at[idx_v.at[a, pl.ds(j * G, G)]],
                rows_v.at[a, pl.ds(j * G, G)],
                gsems[a],
            )

    def drain_gathers(a):
        for j in range(K):
            pltpu.make_async_copy(
                table_hbm.at[pl.ds(0, G)],
                rows_v.at[a, pl.ds(j * G, G)],
                gsems[a],
            ).wait()

    def transpose(a, c):
        def tr_body(d0, carry):
            drot = lax.rem(iota + d0, 32)
            rbase = (drot // 8) * 32 + lax.rem(drot, 8)
            for btl in range(4):
                rvec = rbase + btl * 8
                for g in range(8):
                    bvec = iota + (btl * G + g * 16)
                    v = plsc.load_gather(rows_v.at[a], [bvec, drot])
                    plsc.store_scatter(
                        trans_v.at[c], [rvec, iota + g * 16], v
                    )
            return carry

        lax.fori_loop(0, LATENT_SIZE, tr_body, 0)

    def fire_write(c, t):
        f0 = flat_base + t * KG
        s2 = f0 // B
        bt0 = (f0 % B) // G
        for dt in range(4):
            for btl in range(4):
                pltpu.async_copy(
                    trans_v.at[c, pl.ds(dt * 32 + btl * 8, 8)],
                    out_hbm.at[s2, dt, bt0 + btl],
                    wsems[c],
                )

    def drain_write(c):
        for seg in range(16):
            pltpu.make_async_copy(
                out_hbm.at[0, 0, 0],
                trans_v.at[c, pl.ds(seg * 8, 8)],
                wsems[c],
            ).wait()

    stage_and_fire(0, 0)
    stage_and_fire(1, 1)
    drain_gathers(0)
    transpose(0, 0)
    fire_write(0, 0)
    stage_and_fire(0, 2)
    drain_gathers(1)
    transpose(1, 1)
    fire_write(1, 1)

    def body(p, carry):
        t = 2 * p
        stage_and_fire(1, t + 1)
        drain_gathers(0)
        drain_write(0)
        transpose(0, 0)
        fire_write(0, t)
        stage_and_fire(0, t + 2)
        drain_gathers(1)
        drain_write(1)
        transpose(1, 1)
        fire_write(1, t + 1)
        return carry

    lax.fori_loop(1, n_stages // 2 - 1, body, 0)

    t = n_stages - 2
    stage_and_fire(1, t + 1)
    drain_gathers(0)
    drain_write(0)
    transpose(0, 0)
    fire_write(0, t)
    drain_gathers(1)
    drain_write(1)
    transpose(1, 1)
    fire_write(1, t + 1)
    drain_write(0)
    drain_write(1)


def kernel(x, latent_tdirs):
    xt = jnp.transpose(x, (1, 2, 0)).reshape(S * B).astype(jnp.int32)
    n_stages = (S * B) // KG // NW

    mesh = plsc.VectorSubcoreMesh(core_axis_name="c", subcore_axis_name="s")
    run = functools.partial(
        pl.kernel,
        mesh=mesh,
        compiler_params=pltpu.CompilerParams(
            use_tc_tiling_on_sc=False, needs_layout_passes=False
        ),
        out_type=jax.ShapeDtypeStruct((S, 4, B // G, 8, G), jnp.float32),
        scratch_types=[
            pltpu.VMEM((2, KG), jnp.int32),
            pltpu.VMEM((2, KG, LATENT_SIZE), jnp.float32),
            pltpu.VMEM((2, G, G), jnp.float32),
            pltpu.SemaphoreType.DMA,
            pltpu.SemaphoreType.DMA,
            pltpu.SemaphoreType.DMA,
            pltpu.SemaphoreType.DMA,
        ],
    )(functools.partial(_embed_kernel, n_stages))

    out_t = run(xt, latent_tdirs)
    return jnp.transpose(out_t, (2, 4, 0, 1, 3)).reshape(B, S, LATENT_SIZE)


# speedup vs baseline: 3.5173x; 1.4195x over previous
import functools

import jax
import jax.numpy as jnp
from jax import lax
from jax.experimental import pallas as pl
from jax.experimental.pallas import tpu as pltpu
from jax.experimental.pallas import tpu_sc as plsc

DICT_LEN = 1000000
LATENT_SIZE = 32

S = 200
B = 16384
G = 128
K = 4
KG = K * G

NC = 2
NS = 16
NW = NC * NS


def _embed_kernel(n_stages, idx_hbm, table_hbm, out_hbm,
                  idx_v, rows_v, trans_v, gsem0, gsem1, wsem0, wsem1):
    wid = lax.axis_index("s") * NC + lax.axis_index("c")
    flat_base = wid * n_stages * KG
    gsems = (gsem0, gsem1)
    wsems = (wsem0, wsem1)

    iota = lax.iota(jnp.int32, 16)

    def stage_and_fire(a, t):
        f0 = flat_base + t * KG
        pltpu.sync_copy(idx_hbm.at[pl.ds(f0, KG)], idx_v.at[a])
        for j in range(K):
            pltpu.async_copy(
                table_hbm.at[idx_v.at[a, pl.ds(j * G, G)]],
                rows_v.at[a, pl.ds(j * G, G)],
                gsems[a],
            )

    def drain_gathers(a):
        for j in range(K):
            pltpu.make_async_copy(
                table_hbm.at[pl.ds(0, G)],
                rows_v.at[a, pl.ds(j * G, G)],
                gsems[a],
            ).wait()

    def transpose(a, c):
        def tr_body(d0, carry):
            drot = lax.rem(iota + d0, 32)
            rbase = (drot // 8) * 32 + lax.rem(drot, 8)
            for btl in range(4):
                rvec = rbase + btl * 8
                vs = []
                for g in range(8):
                    bvec = iota + (btl * G + g * 16)
                    vs.append(plsc.load_gather(rows_v.at[a], [bvec, drot]))
                for g in range(8):
                    plsc.store_scatter(
                        trans_v.at[c], [rvec, iota + g * 16], vs[g]
                    )
            return carry

        lax.fori_loop(0, LATENT_SIZE, tr_body, 0)

    def fire_write(c, t):
        f0 = flat_base + t * KG
        s2 = f0 // B
        bt0 = (f0 % B) // G
        for dt in range(4):
            for btl in range(4):
                pltpu.async_copy(
                    trans_v.at[c, pl.ds(dt * 32 + btl * 8, 8)],
                    out_hbm.at[s2, dt, bt0 + btl],
                    wsems[c],
                )

    def drain_write(c):
        for seg in range(16):
            pltpu.make_async_copy(
                out_hbm.at[0, 0, 0],
                trans_v.at[c, pl.ds(seg * 8, 8)],
                wsems[c],
            ).wait()

    stage_and_fire(0, 0)
    stage_and_fire(1, 1)
    drain_gathers(0)
    transpose(0, 0)
    fire_write(0, 0)
    stage_and_fire(0, 2)
    drain_gathers(1)
    transpose(1, 1)
    fire_write(1, 1)

    def body(p, carry):
        t = 2 * p
        stage_and_fire(1, t + 1)
        drain_gathers(0)
        drain_write(0)
        transpose(0, 0)
        fire_write(0, t)
        stage_and_fire(0, t + 2)
        drain_gathers(1)
        drain_write(1)
        transpose(1, 1)
        fire_write(1, t + 1)
        return carry

    lax.fori_loop(1, n_stages // 2 - 1, body, 0)

    t = n_stages - 2
    stage_and_fire(1, t + 1)
    drain_gathers(0)
    drain_write(0)
    transpose(0, 0)
    fire_write(0, t)
    drain_gathers(1)
    drain_write(1)
    transpose(1, 1)
    fire_write(1, t + 1)
    drain_write(0)
    drain_write(1)


def kernel(x, latent_tdirs):
    xt = jnp.transpose(x, (1, 2, 0)).reshape(S * B).astype(jnp.int32)
    n_stages = (S * B) // KG // NW

    mesh = plsc.VectorSubcoreMesh(core_axis_name="c", subcore_axis_name="s")
    run = functools.partial(
        pl.kernel,
        mesh=mesh,
        compiler_params=pltpu.CompilerParams(
            use_tc_tiling_on_sc=False, needs_layout_passes=False
        ),
        out_type=jax.ShapeDtypeStruct((S, 4, B // G, 8, G), jnp.float32),
        scratch_types=[
            pltpu.VMEM((2, KG), jnp.int32),
            pltpu.VMEM((2, KG, LATENT_SIZE), jnp.float32),
            pltpu.VMEM((2, G, G), jnp.float32),
            pltpu.SemaphoreType.DMA,
            pltpu.SemaphoreType.DMA,
            pltpu.SemaphoreType.DMA,
            pltpu.SemaphoreType.DMA,
        ],
    )(functools.partial(_embed_kernel, n_stages))

    out_t = run(xt, latent_tdirs)
    return jnp.transpose(out_t, (2, 4, 0, 1, 3)).reshape(B, S, LATENT_SIZE)
